# baseline (speedup 1.0000x reference)
import functools

import jax
import jax.numpy as jnp
from jax import lax
from jax.experimental import pallas as pl
from jax.experimental.pallas import tpu as pltpu
from jax.experimental.pallas import tpu_sc as plsc

N = 10000
M = 320000
NE = 20000
D = 128
ALPHA = 0.5

NC, NS = 2, 16
NW = NC * NS
CPW = D // NW
CHUNK = 1280
GROUPS = CHUNK // 16
NCHUNKS = M // CHUNK
DEG_PER = -(-N // NW)
DEG_PAD = 320


def _seg_sum_body(compute_deg, n_rows, n_segs, src_hbm, seg_hbm, table_hbm,
                  *refs):
    if compute_deg:
        (out_hbm, deg_hbm, table_v, acc_v, deg_v,
         sb0, gb0, sb1, gb1, sem0, sem1) = refs
    else:
        (out_hbm, table_v, acc_v, sb0, gb0, sb1, gb1, sem0, sem1) = refs
        deg_hbm = deg_v = None
    w = lax.axis_index("s") * NC + lax.axis_index("c")

    pltpu.sync_copy(table_hbm.at[w], table_v)

    zero16 = jnp.zeros((16,), jnp.float32)

    def zero_body(i, _):
        acc_v[pl.ds(i * 16, 16)] = zero16
        return 0
    lax.fori_loop(0, (n_segs * CPW) // 16, zero_body, 0)

    if compute_deg:
        def zero_deg(i, _):
            deg_v[pl.ds(i * 16, 16)] = zero16
            return 0
        lax.fori_loop(0, DEG_PAD // 16, zero_deg, 0)
        deg_lo = w * DEG_PER
        ones16 = jnp.ones((16,), jnp.float32)

    bufs = ((sb0, gb0, sem0), (sb1, gb1, sem1))

    def issue(ci, b):
        sb, gb, sem = bufs[b]
        pltpu.async_copy(src_hbm.at[pl.ds(ci * CHUNK, CHUNK)], sb, sem)
        pltpu.async_copy(seg_hbm.at[pl.ds(ci * CHUNK, CHUNK)], gb, sem)

    def wait(b):
        sb, gb, sem = bufs[b]
        pltpu.make_async_copy(src_hbm.at[pl.ds(0, CHUNK)], sb, sem).wait()
        pltpu.make_async_copy(seg_hbm.at[pl.ds(0, CHUNK)], gb, sem).wait()

    def consume(b):
        sb, gb, _ = bufs[b]

        def grp(g, _):
            i16 = sb[pl.ds(g * 16, 16)]
            o16 = gb[pl.ds(g * 16, 16)]
            vals = []
            for p in range(CPW // 2):
                word = plsc.bitcast(
                    plsc.load_gather(table_v, [i16 + p * n_rows]), jnp.int32)
                vals.append(plsc.bitcast(word << 16, jnp.float32))
                vals.append(plsc.bitcast(
                    word & jnp.int32(-65536), jnp.float32))
            for c in range(CPW):
                plsc.addupdate_scatter(acc_v, [o16 + c * n_segs], vals[c])
            if compute_deg:
                mk = (i16 >= deg_lo) & (i16 < deg_lo + DEG_PER)
                di = jnp.where(mk, i16 - deg_lo, 0)
                plsc.addupdate_scatter(deg_v, [di], ones16, mask=mk)
            return 0
        lax.fori_loop(0, GROUPS, grp, 0)

    issue(0, 0)

    def pair(p, _):
        issue(2 * p + 1, 1)
        wait(0)
        consume(0)
        issue(jnp.minimum(2 * p + 2, NCHUNKS - 1), 0)
        wait(1)
        consume(1)
        return 0
    lax.fori_loop(0, NCHUNKS // 2, pair, 0)
    wait(0)

    pltpu.sync_copy(acc_v, out_hbm.at[w])
    if compute_deg:
        pltpu.sync_copy(deg_v, deg_hbm.at[w])


def _seg_sum_sc(src, seg, table_blocked, n_rows, n_segs, compute_deg):
    mesh = plsc.VectorSubcoreMesh(core_axis_name="c", subcore_axis_name="s")
    out_type = [jax.ShapeDtypeStruct((NW, n_segs * CPW), jnp.float32)]
    scratch = [
        pltpu.VMEM((n_rows * CPW // 2,), jnp.float32),
        pltpu.VMEM((n_segs * CPW,), jnp.float32),
    ]
    if compute_deg:
        out_type.append(jax.ShapeDtypeStruct((NW, DEG_PAD), jnp.float32))
        scratch.append(pltpu.VMEM((DEG_PAD,), jnp.float32))
    scratch += [
        pltpu.VMEM((CHUNK,), jnp.int32),
        pltpu.VMEM((CHUNK,), jnp.int32),
        pltpu.VMEM((CHUNK,), jnp.int32),
        pltpu.VMEM((CHUNK,), jnp.int32),
        pltpu.SemaphoreType.DMA,
        pltpu.SemaphoreType.DMA,
    ]
    fn = pl.kernel(
        functools.partial(_seg_sum_body, compute_deg, n_rows, n_segs),
        out_type=tuple(out_type),
        mesh=mesh,
        scratch_types=tuple(scratch),
        compiler_params=pltpu.CompilerParams(needs_layout_passes=False),
    )
    return fn(src, seg, table_blocked)


def _pack_bf16_cm(x_cm):
    d, n = x_cm.shape
    pairs = x_cm.astype(jnp.bfloat16).reshape(d // 2, 2, n).transpose(0, 2, 1)
    return jax.lax.bitcast_convert_type(pairs, jnp.float32).reshape(NW, -1)


def _mm_t_body(x_ref, w_ref, b_ref, o_ref):
    o_ref[...] = jnp.dot(x_ref[...], w_ref[...],
                         preferred_element_type=jnp.float32).T + b_ref[...]


def _tc_matmul_t(x, w, b_col):
    n, k = x.shape
    cols = w.shape[1]
    return pl.pallas_call(
        _mm_t_body,
        out_shape=jax.ShapeDtypeStruct((cols, n), jnp.float32),
    )(x, w, b_col.reshape(cols, 1))


def _mm_cm_body(w_ref, x_ref, o_ref):
    o_ref[...] = jnp.dot(w_ref[...], x_ref[...],
                         preferred_element_type=jnp.float32)


def _tc_matmul_cm(w_t, x_cm):
    d = w_t.shape[0]
    n = x_cm.shape[1]
    return pl.pallas_call(
        _mm_cm_body,
        out_shape=jax.ShapeDtypeStruct((d, n), jnp.float32),
    )(w_t, x_cm)


def _final_body(a_ref, c_ref, x_ref, deg_ref, w2b_ref, ww_ref, wb_ref, o_ref):
    xv = deg_ref[...] * (a_ref[...].T + w2b_ref[...]) + c_ref[...].T
    xn = (1.0 - ALPHA) * xv + ALPHA * x_ref[...]
    o_ref[...] = jnp.dot(xn, ww_ref[...],
                         preferred_element_type=jnp.float32) + wb_ref[...]


def _tc_final(a_cm, c_cm, x, deg, w2_b, w_w, w_b):
    n = x.shape[0]
    return pl.pallas_call(
        _final_body,
        out_shape=jax.ShapeDtypeStruct((n, D), jnp.float32),
    )(a_cm, c_cm, x, deg.reshape(n, 1), w2_b.reshape(1, D), w_w,
      w_b.reshape(1, D))


def kernel(X, vertex, edges, W1_w, W1_b, W2_w, W2_b, W_w, W_b):
    vertex = vertex.astype(jnp.int32)
    edges = edges.astype(jnp.int32)

    w_cat = jnp.concatenate([W1_w, W2_w[:D]], axis=1)
    b_cat = jnp.concatenate([W1_b, jnp.zeros((D,), jnp.float32)])
    xw_cm = _tc_matmul_t(X, w_cat, b_cat)
    h_blocked = _pack_bf16_cm(xw_cm[:D])
    a_cm = xw_cm[D:]

    xe_b, deg_b = _seg_sum_sc(vertex, edges, h_blocked, N, NE,
                              compute_deg=True)
    deg = deg_b[:, :DEG_PER].reshape(-1)[:N]

    b_cm = _tc_matmul_cm(W2_w[D:].T, xe_b.reshape(D, NE))

    (c_b,) = _seg_sum_sc(edges, vertex, _pack_bf16_cm(b_cm), NE, N,
                         compute_deg=False)

    return _tc_final(a_cm, c_b.reshape(D, N), X, deg, W2_b, W_w, W_b)


# speedup vs baseline: 1.1684x; 1.1684x over previous
import functools

import jax
import jax.numpy as jnp
from jax import lax
from jax.experimental import pallas as pl
from jax.experimental.pallas import tpu as pltpu
from jax.experimental.pallas import tpu_sc as plsc

N = 10000
M = 320000
NE = 20000
D = 128
ALPHA = 0.5

NC, NS = 2, 16
NW = NC * NS
CPW = D // NW
CHUNK = 1600
GROUPS = CHUNK // 16
NCHUNKS = M // CHUNK
DEG_PER = -(-N // NW)
DEG_PAD = 320


def _seg_sum_body(compute_deg, n_rows, n_segs, src_hbm, seg_hbm, table_hbm,
                  *refs):
    if compute_deg:
        (out_hbm, deg_hbm, table_v, acc_v, deg_v,
         sb0, gb0, sb1, gb1, sem0, sem1) = refs
    else:
        (out_hbm, table_v, acc_v, sb0, gb0, sb1, gb1, sem0, sem1) = refs
        deg_hbm = deg_v = None
    w = lax.axis_index("s") * NC + lax.axis_index("c")

    pltpu.sync_copy(table_hbm.at[w], table_v)

    zero16 = jnp.zeros((16,), jnp.float32)

    def zero_body(i, _):
        acc_v[pl.ds(i * 16, 16)] = zero16
        return 0
    lax.fori_loop(0, (n_segs * CPW) // 16, zero_body, 0)

    if compute_deg:
        def zero_deg(i, _):
            deg_v[pl.ds(i * 16, 16)] = zero16
            return 0
        lax.fori_loop(0, DEG_PAD // 16, zero_deg, 0)
        deg_lo = w * DEG_PER
        ones16 = jnp.ones((16,), jnp.float32)

    bufs = ((sb0, gb0, sem0), (sb1, gb1, sem1))

    def issue(ci, b):
        sb, gb, sem = bufs[b]
        pltpu.async_copy(src_hbm.at[pl.ds(ci * CHUNK, CHUNK)], sb, sem)
        pltpu.async_copy(seg_hbm.at[pl.ds(ci * CHUNK, CHUNK)], gb, sem)

    def wait(b):
        sb, gb, sem = bufs[b]
        pltpu.make_async_copy(src_hbm.at[pl.ds(0, CHUNK)], sb, sem).wait()
        pltpu.make_async_copy(seg_hbm.at[pl.ds(0, CHUNK)], gb, sem).wait()

    def consume(b):
        sb, gb, _ = bufs[b]

        def grp(g, _):
            i16 = sb[pl.ds(g * 16, 16)]
            o16 = gb[pl.ds(g * 16, 16)]
            vals = [plsc.load_gather(table_v, [i16 + c * n_rows])
                    for c in range(CPW)]
            for c in range(CPW):
                plsc.addupdate_scatter(acc_v, [o16 + c * n_segs], vals[c])
            if compute_deg:
                mk = (i16 >= deg_lo) & (i16 < deg_lo + DEG_PER)
                di = jnp.where(mk, i16 - deg_lo, 0)
                plsc.addupdate_scatter(deg_v, [di], ones16, mask=mk)
            return 0
        lax.fori_loop(0, GROUPS, grp, 0)

    issue(0, 0)

    def pair(p, _):
        issue(2 * p + 1, 1)
        wait(0)
        consume(0)
        issue(jnp.minimum(2 * p + 2, NCHUNKS - 1), 0)
        wait(1)
        consume(1)
        return 0
    lax.fori_loop(0, NCHUNKS // 2, pair, 0)
    wait(0)

    pltpu.sync_copy(acc_v, out_hbm.at[w])
    if compute_deg:
        pltpu.sync_copy(deg_v, deg_hbm.at[w])


def _seg_sum_sc(src, seg, table_blocked, n_rows, n_segs, compute_deg):
    mesh = plsc.VectorSubcoreMesh(core_axis_name="c", subcore_axis_name="s")
    out_type = [jax.ShapeDtypeStruct((NW, n_segs * CPW), jnp.float32)]
    scratch = [
        pltpu.VMEM((n_rows * CPW,), jnp.float32),
        pltpu.VMEM((n_segs * CPW,), jnp.float32),
    ]
    if compute_deg:
        out_type.append(jax.ShapeDtypeStruct((NW, DEG_PAD), jnp.float32))
        scratch.append(pltpu.VMEM((DEG_PAD,), jnp.float32))
    scratch += [
        pltpu.VMEM((CHUNK,), jnp.int32),
        pltpu.VMEM((CHUNK,), jnp.int32),
        pltpu.VMEM((CHUNK,), jnp.int32),
        pltpu.VMEM((CHUNK,), jnp.int32),
        pltpu.SemaphoreType.DMA,
        pltpu.SemaphoreType.DMA,
    ]
    fn = pl.kernel(
        functools.partial(_seg_sum_body, compute_deg, n_rows, n_segs),
        out_type=tuple(out_type),
        mesh=mesh,
        scratch_types=tuple(scratch),
        compiler_params=pltpu.CompilerParams(needs_layout_passes=False),
    )
    return fn(src, seg, table_blocked)


def _mm_t_body(x_ref, w_ref, b_ref, o_ref):
    o_ref[...] = jnp.dot(x_ref[...], w_ref[...],
                         preferred_element_type=jnp.float32).T + b_ref[...]


def _tc_matmul_t(x, w, b_col):
    n, k = x.shape
    cols = w.shape[1]
    return pl.pallas_call(
        _mm_t_body,
        out_shape=jax.ShapeDtypeStruct((cols, n), jnp.float32),
    )(x, w, b_col.reshape(cols, 1))


def _mm_cm_body(w_ref, x_ref, o_ref):
    o_ref[...] = jnp.dot(w_ref[...], x_ref[...],
                         preferred_element_type=jnp.float32)


def _tc_matmul_cm(w_t, x_cm):
    d = w_t.shape[0]
    n = x_cm.shape[1]
    return pl.pallas_call(
        _mm_cm_body,
        out_shape=jax.ShapeDtypeStruct((d, n), jnp.float32),
    )(w_t, x_cm)


def _final_body(a_ref, c_ref, x_ref, deg_ref, w2b_ref, ww_ref, wb_ref, o_ref):
    xv = deg_ref[...] * (a_ref[...].T + w2b_ref[...]) + c_ref[...].T
    xn = (1.0 - ALPHA) * xv + ALPHA * x_ref[...]
    o_ref[...] = jnp.dot(xn, ww_ref[...],
                         preferred_element_type=jnp.float32) + wb_ref[...]


def _tc_final(a_cm, c_cm, x, deg, w2_b, w_w, w_b):
    n = x.shape[0]
    return pl.pallas_call(
        _final_body,
        out_shape=jax.ShapeDtypeStruct((n, D), jnp.float32),
    )(a_cm, c_cm, x, deg.reshape(n, 1), w2_b.reshape(1, D), w_w,
      w_b.reshape(1, D))


def kernel(X, vertex, edges, W1_w, W1_b, W2_w, W2_b, W_w, W_b):
    vertex = vertex.astype(jnp.int32)
    edges = edges.astype(jnp.int32)

    w_cat = jnp.concatenate([W1_w, W2_w[:D]], axis=1)
    b_cat = jnp.concatenate([W1_b, jnp.zeros((D,), jnp.float32)])
    xw_cm = _tc_matmul_t(X, w_cat, b_cat)
    h_blocked = xw_cm[:D].reshape(NW, CPW * N)
    a_cm = xw_cm[D:]

    xe_b, deg_b = _seg_sum_sc(vertex, edges, h_blocked, N, NE,
                              compute_deg=True)
    deg = deg_b[:, :DEG_PER].reshape(-1)[:N]

    b_cm = _tc_matmul_cm(W2_w[D:].T, xe_b.reshape(D, NE))

    (c_b,) = _seg_sum_sc(edges, vertex, b_cm.reshape(NW, CPW * NE), NE, N,
                         compute_deg=False)

    return _tc_final(a_cm, c_b.reshape(D, N), X, deg, W2_b, W_w, W_b)
